# Initial kernel scaffold; baseline (speedup 1.0000x reference)
import jax
import jax.numpy as jnp
from jax.experimental import pallas as pl


def kernel(adjs, rels, vents, entlens, renc, params):
    raise NotImplementedError("write your pallas kernel here")


# speedup vs baseline: 30.0666x; 30.0666x over previous
import functools
import math

import jax
import jax.numpy as jnp
from jax.experimental import pallas as pl

_B = 4
_E = 192
_R = 64
_N = _E + _R
_HSZ = 512
_RTOKS = 1000
_PROP = 2
_H = 4
_DH = _HSZ // _H


def _dot(a, b):
    return jax.lax.dot_general(
        a, b, (((1,), (0,)), ((), ())), preferred_element_type=jnp.float32
    )


def _layernorm(x, g, b, eps=1e-5):
    m = jnp.mean(x, axis=-1, keepdims=True)
    xc = x - m
    v = jnp.mean(xc * xc, axis=-1, keepdims=True)
    return xc * jax.lax.rsqrt(v + eps) * g + b


def _graph_kernel(
    rels_ref, vents_ref, adjs_ref, renc_ref,
    wq_ref, wk_ref, wv_ref, wo_ref,
    w1_ref, b1_ref, w2_ref, b2_ref, a1_ref,
    ln1g_ref, ln1b_ref, ln2g_ref, ln2b_ref,
    out_ref,
):
    rels = rels_ref[0]
    ids = jnp.broadcast_to(rels.reshape(_R, 1), (_R, _RTOKS))
    iota = jax.lax.broadcasted_iota(jnp.int32, (_R, _RTOKS), 1)
    onehot = (ids == iota).astype(jnp.float32)
    vrel = _dot(onehot, renc_ref[...])

    vgraph = jnp.concatenate([vents_ref[0], vrel], axis=0)
    masked = adjs_ref[0] == 0.0
    scale = 1.0 / math.sqrt(_DH)

    for j in range(_PROP):
        q = _dot(vgraph, wq_ref[j])
        k = _dot(vgraph, wk_ref[j])
        v = _dot(vgraph, wv_ref[j])
        outs = []
        for h in range(_H):
            sl = slice(h * _DH, (h + 1) * _DH)
            s = _dot(q[:, sl], k[:, sl].T) * scale
            s = jnp.where(masked, -1e9, s)
            s = s - jnp.max(s, axis=-1, keepdims=True)
            e = jnp.exp(s)
            a = e / jnp.sum(e, axis=-1, keepdims=True)
            outs.append(_dot(a, v[:, sl]))
        o = jnp.concatenate(outs, axis=-1)
        attn = _dot(o, wo_ref[j])
        t = _layernorm(attn, ln1g_ref[j], ln1b_ref[j])
        hdn = _dot(t, w1_ref[j]) + b1_ref[j]
        hdn = jnp.where(hdn >= 0.0, hdn, a1_ref[j] * hdn)
        y = _dot(hdn, w2_ref[j]) + b2_ref[j]
        vgraph = _layernorm(y + t, ln2g_ref[j], ln2b_ref[j])

    out_ref[0] = vgraph


@jax.jit
def _run(adjs, rels, vents, renc, stacked):
    rels3 = rels.astype(jnp.int32).reshape(_B, 1, _R)
    rep2 = lambda i: (0, 0)
    rep3 = lambda i: (0, 0, 0)
    in_specs = [
        pl.BlockSpec((1, 1, _R), lambda i: (i, 0, 0)),
        pl.BlockSpec((1, _E, _HSZ), lambda i: (i, 0, 0)),
        pl.BlockSpec((1, _N, _N), lambda i: (i, 0, 0)),
        pl.BlockSpec((_RTOKS, _HSZ), rep2),
        pl.BlockSpec((_PROP, _HSZ, _HSZ), rep3),
        pl.BlockSpec((_PROP, _HSZ, _HSZ), rep3),
        pl.BlockSpec((_PROP, _HSZ, _HSZ), rep3),
        pl.BlockSpec((_PROP, _HSZ, _HSZ), rep3),
        pl.BlockSpec((_PROP, _HSZ, 4 * _HSZ), rep3),
        pl.BlockSpec((_PROP, 4 * _HSZ), rep2),
        pl.BlockSpec((_PROP, 4 * _HSZ, _HSZ), rep3),
        pl.BlockSpec((_PROP, _HSZ), rep2),
        pl.BlockSpec((_PROP, 4 * _HSZ), rep2),
        pl.BlockSpec((_PROP, _HSZ), rep2),
        pl.BlockSpec((_PROP, _HSZ), rep2),
        pl.BlockSpec((_PROP, _HSZ), rep2),
        pl.BlockSpec((_PROP, _HSZ), rep2),
    ]
    gents = pl.pallas_call(
        _graph_kernel,
        grid=(_B,),
        in_specs=in_specs,
        out_specs=pl.BlockSpec((1, _N, _HSZ), lambda i: (i, 0, 0)),
        out_shape=jax.ShapeDtypeStruct((_B, _N, _HSZ), jnp.float32),
    )(rels3, vents, adjs, renc, *stacked)
    return gents


def kernel(adjs, rels, vents, entlens, renc, params):
    fields = ['Wq', 'Wk', 'Wv', 'Wo', 'W1', 'b1', 'W2', 'b2', 'a1',
              'ln1_g', 'ln1_b', 'ln2_g', 'ln2_b']
    stacked = [jnp.stack([p[f] for p in params], axis=0) for f in fields]
    gents = _run(adjs, rels, vents, renc, stacked)
    globv = gents[:, _E, :]
    emask = jnp.arange(_N)[None, :] <= entlens[:, None]
    return (globv, gents, emask)


# speedup vs baseline: 46.0719x; 1.5323x over previous
import math

import jax
import jax.numpy as jnp
from jax.experimental import pallas as pl

_B = 4
_E = 192
_R = 64
_N = _E + _R
_HSZ = 512
_RTOKS = 1000
_PROP = 2
_H = 4
_DH = _HSZ // _H

_FIELDS = ['Wq', 'Wk', 'Wv', 'Wo', 'W1', 'b1', 'W2', 'b2', 'a1',
           'ln1_g', 'ln1_b', 'ln2_g', 'ln2_b']


def _dot(a, b):
    return jax.lax.dot_general(
        a, b, (((1,), (0,)), ((), ())), preferred_element_type=jnp.float32
    )


def _layernorm(x, g, b, eps=1e-5):
    m = jnp.mean(x, axis=-1, keepdims=True)
    xc = x - m
    v = jnp.mean(xc * xc, axis=-1, keepdims=True)
    return xc * jax.lax.rsqrt(v + eps) * g + b


def _graph_kernel(rels_ref, vents_ref, adjs_ref, renc_ref, *refs):
    out_ref = refs[-1]
    prefs = refs[:-1]

    rels = rels_ref[0]
    ids = jnp.broadcast_to(rels.reshape(_R, 1), (_R, _RTOKS))
    iota = jax.lax.broadcasted_iota(jnp.int32, (_R, _RTOKS), 1)
    onehot = (ids == iota).astype(jnp.float32)
    vrel = _dot(onehot, renc_ref[...])

    vgraph = jnp.concatenate([vents_ref[0], vrel], axis=0)
    masked = adjs_ref[0] == 0.0
    scale = 1.0 / math.sqrt(_DH)

    nf = len(_FIELDS)
    for j in range(_PROP):
        p = dict(zip(_FIELDS, prefs[j * nf:(j + 1) * nf]))
        q = _dot(vgraph, p['Wq'][...])
        k = _dot(vgraph, p['Wk'][...])
        v = _dot(vgraph, p['Wv'][...])
        outs = []
        for h in range(_H):
            sl = slice(h * _DH, (h + 1) * _DH)
            s = _dot(q[:, sl], k[:, sl].T) * scale
            s = jnp.where(masked, -1e9, s)
            s = s - jnp.max(s, axis=-1, keepdims=True)
            e = jnp.exp(s)
            a = e / jnp.sum(e, axis=-1, keepdims=True)
            outs.append(_dot(a, v[:, sl]))
        o = jnp.concatenate(outs, axis=-1)
        attn = _dot(o, p['Wo'][...])
        t = _layernorm(attn, p['ln1_g'][0], p['ln1_b'][0])
        hdn = _dot(t, p['W1'][...]) + p['b1'][0]
        hdn = jnp.where(hdn >= 0.0, hdn, p['a1'][0] * hdn)
        y = _dot(hdn, p['W2'][...]) + p['b2'][0]
        vgraph = _layernorm(y + t, p['ln2_g'][0], p['ln2_b'][0])

    out_ref[0] = vgraph


@jax.jit
def _run(adjs, rels, vents, entlens, renc, params):
    rels3 = rels.astype(jnp.int32).reshape(_B, 1, _R)
    rep2 = lambda i: (0, 0)

    flat = []
    in_specs = [
        pl.BlockSpec((1, 1, _R), lambda i: (i, 0, 0)),
        pl.BlockSpec((1, _E, _HSZ), lambda i: (i, 0, 0)),
        pl.BlockSpec((1, _N, _N), lambda i: (i, 0, 0)),
        pl.BlockSpec((_RTOKS, _HSZ), rep2),
    ]
    for j in range(_PROP):
        for f in _FIELDS:
            w = params[j][f]
            if w.ndim == 1:
                w = w.reshape(1, -1)
            flat.append(w)
            in_specs.append(pl.BlockSpec(w.shape, rep2))

    gents = pl.pallas_call(
        _graph_kernel,
        grid=(_B,),
        in_specs=in_specs,
        out_specs=pl.BlockSpec((1, _N, _HSZ), lambda i: (i, 0, 0)),
        out_shape=jax.ShapeDtypeStruct((_B, _N, _HSZ), jnp.float32),
    )(rels3, vents, adjs, renc, *flat)

    globv = gents[:, _E, :]
    emask = jnp.arange(_N)[None, :] <= entlens[:, None]
    return (globv, gents, emask)


def kernel(adjs, rels, vents, entlens, renc, params):
    return _run(adjs, rels, vents, entlens, renc, params)
